# baseline (speedup 1.0000x reference)
import functools

import jax
import jax.numpy as jnp
from jax import lax
from jax.experimental import pallas as pl
from jax.experimental.pallas import tpu as pltpu
from jax.experimental.pallas import tpu_sc as plsc

F32 = jnp.float32
I32 = jnp.int32

NW = 32
LANES = 16
SUP = 128
TC_BLK = 1280


def _proj_body(y_ref, w_ref, as_ref, ad_ref, b_ref, h_ref, s_ref, d_ref,
               *, kdim, relu, bias):
    y = y_ref[...]
    if bias:
        y = y + b_ref[...]
    if relu:
        y = jnp.maximum(y, 0.0)
    if kdim == 1:
        h = y * w_ref[...]
    else:
        h = jnp.dot(y, w_ref[...], preferred_element_type=F32)
    h_ref[...] = h
    s_ref[...] = jnp.sum(h * as_ref[...], axis=1, keepdims=True)
    d_ref[...] = jnp.sum(h * ad_ref[...], axis=1, keepdims=True)


def _proj(y, w, a_s, a_d, bias, relu):
    np_, k = y.shape
    m = w.shape[1]
    b2 = (bias if bias is not None else jnp.zeros((k,), F32)).reshape(1, k)
    body = functools.partial(_proj_body, kdim=k, relu=relu,
                             bias=bias is not None)
    return pl.pallas_call(
        body,
        grid=(np_ // TC_BLK,),
        in_specs=[
            pl.BlockSpec((TC_BLK, k), lambda i: (i, 0)),
            pl.BlockSpec((k, m), lambda i: (0, 0)),
            pl.BlockSpec((1, m), lambda i: (0, 0)),
            pl.BlockSpec((1, m), lambda i: (0, 0)),
            pl.BlockSpec((1, k), lambda i: (0, 0)),
        ],
        out_specs=[
            pl.BlockSpec((TC_BLK, m), lambda i: (i, 0)),
            pl.BlockSpec((TC_BLK, 1), lambda i: (i, 0)),
            pl.BlockSpec((TC_BLK, 1), lambda i: (i, 0)),
        ],
        out_shape=[
            jax.ShapeDtypeStruct((np_, m), F32),
            jax.ShapeDtypeStruct((np_, 1), F32),
            jax.ShapeDtypeStruct((np_, 1), F32),
        ],
    )(y, w, a_s.reshape(1, m), a_d.reshape(1, m), b2)


def _lin1_body(x_ref, w_ref, b_ref, o_ref):
    o_ref[...] = x_ref[...] * w_ref[...] + b_ref[...]


def _lin1(x, w, b):
    np_ = x.shape[0]
    m = w.shape[1]
    return pl.pallas_call(
        _lin1_body,
        grid=(np_ // TC_BLK,),
        in_specs=[
            pl.BlockSpec((TC_BLK, 1), lambda i: (i, 0)),
            pl.BlockSpec((1, m), lambda i: (0, 0)),
            pl.BlockSpec((1, m), lambda i: (0, 0)),
        ],
        out_specs=pl.BlockSpec((TC_BLK, m), lambda i: (i, 0)),
        out_shape=jax.ShapeDtypeStruct((np_, m), F32),
    )(x, w, b.reshape(1, m))


def _mid_body(m_ref, bg_ref, w_ref, b2_ref, x1_ref, o_ref):
    t = m_ref[...] + bg_ref[...]
    t = jnp.dot(t, w_ref[...], preferred_element_type=F32) + b2_ref[...]
    t = t + x1_ref[...]
    o_ref[...] = jnp.where(t >= 0, t, 0.01 * t)


def _fin_body(m_ref, bg_ref, w_ref, b3_ref, x2_ref, o_ref):
    t = m_ref[...] + bg_ref[...]
    t = jnp.dot(t, w_ref[...], preferred_element_type=F32) + b3_ref[...]
    o_ref[...] = jax.nn.sigmoid(x2_ref[...] + t)


def _resid(body, msg, bg, w, b, xr):
    np_, k = msg.shape
    m = w.shape[1]
    return pl.pallas_call(
        body,
        grid=(np_ // TC_BLK,),
        in_specs=[
            pl.BlockSpec((TC_BLK, k), lambda i: (i, 0)),
            pl.BlockSpec((1, k), lambda i: (0, 0)),
            pl.BlockSpec((k, m), lambda i: (0, 0)),
            pl.BlockSpec((1, m), lambda i: (0, 0)),
            pl.BlockSpec((TC_BLK, m), lambda i: (i, 0)),
        ],
        out_specs=pl.BlockSpec((TC_BLK, m), lambda i: (i, 0)),
        out_shape=jax.ShapeDtypeStruct((np_, m), F32),
    )(msg, bg.reshape(1, k), w, b.reshape(1, m), xr)


@functools.lru_cache(maxsize=None)
def _make_sc_gat(np_, d, ep, npt, e_real):
    mesh = plsc.VectorSubcoreMesh(core_axis_name="c", subcore_axis_name="s")
    dch = d // LANES

    @functools.partial(
        pl.kernel,
        mesh=mesh,
        out_type=jax.ShapeDtypeStruct((np_, d), F32),
        compiler_params=pltpu.CompilerParams(needs_layout_passes=False),
        scratch_types=[
            pltpu.VMEM((np_,), F32),
            pltpu.VMEM((npt,), F32),
            pltpu.VMEM((npt + LANES,), F32),
            pltpu.VMEM((npt + 1, d), F32),
            pltpu.VMEM(((SUP + 8) * LANES,), I32),
            pltpu.VMEM(((SUP + 8) * LANES,), I32),
            pltpu.VMEM((LANES, d), F32),
            pltpu.VMEM((LANES, d), F32),
            pltpu.VMEM((LANES, d), F32),
            pltpu.VMEM((LANES, d), F32),
            pltpu.VMEM((LANES,), I32),
            pltpu.VMEM((LANES,), F32),
            pltpu.VMEM((2 * NW + LANES,), I32),
            pltpu.SemaphoreType.DMA,
            pltpu.SemaphoreType.DMA,
            pltpu.SemaphoreType.DMA,
            pltpu.SemaphoreType.DMA,
        ],
    )
    def sc_gat(h_hbm, s_hbm, d_hbm, srcs_hbm, dsts_hbm, bounds_hbm,
               out_hbm, s_all, d_loc, den, acc, st_src, st_dst,
               rows0, rows1, rows2, rows3, sc_i, sc_f, bounds_v,
               sem0, sem1, sem2, sem3):
        w = lax.axis_index("s") * 2 + lax.axis_index("c")
        base_node = w * npt
        pltpu.sync_copy(bounds_hbm, bounds_v)
        cstart = bounds_v[pl.ds(w, LANES)][0]
        cend = bounds_v[pl.ds(NW + w, LANES)][0]
        pltpu.sync_copy(s_hbm, s_all)
        pltpu.sync_copy(d_hbm.at[pl.ds(base_node, npt)], d_loc)

        iota = lax.iota(I32, LANES)
        zero16 = jnp.zeros((LANES,), F32)

        def redm(i, m):
            return jnp.maximum(m, s_all[pl.ds(i * LANES, LANES)])
        mvec = lax.fori_loop(0, np_ // LANES, redm,
                             jnp.full((LANES,), -3.4e38, F32))
        smax = jnp.max(mvec)

        def zden(i, _):
            den[pl.ds(i * LANES, LANES)] = zero16
            return 0
        lax.fori_loop(0, (npt + LANES) // LANES, zden, 0)

        def zacc(r, _):
            for c in range(dch):
                acc[r, pl.ds(c * LANES, LANES)] = zero16
            return 0
        lax.fori_loop(0, npt + 1, zacc, 0)

        def edge_vecs(ch, o):
            sv = st_src[pl.ds(o * LANES, LANES)]
            dv = st_dst[pl.ds(o * LANES, LANES)]
            eid = ch * LANES + iota
            dlr = dv - base_node
            valid = (dlr >= 0) & (dlr < npt) & (eid < e_real)
            dl0 = jnp.where(valid, dlr, 0)
            ssrc = plsc.load_gather(s_all, [sv])
            dd = plsc.load_gather(d_loc, [dl0])
            mm = jnp.maximum(smax + dd, 0.0)
            t = ssrc + dd
            e = jnp.where(t >= 0, t, 0.2 * t)
            ex = jnp.where(valid, jnp.exp(e - mm), 0.0)
            return sv, valid, dlr, dl0, ex

        def sup1(j, _):
            bch = cstart + j * SUP
            nch = jnp.minimum(SUP, cend - bch)
            pltpu.sync_copy(
                srcs_hbm.at[pl.ds(bch * LANES, (SUP + 8) * LANES)], st_src)
            pltpu.sync_copy(
                dsts_hbm.at[pl.ds(bch * LANES, (SUP + 8) * LANES)], st_dst)

            def ch1(o, _):
                _, valid, dlr, _, ex = edge_vecs(bch + o, o)
                dl = jnp.where(valid, dlr, npt)
                pcs = plsc.cumsum(ex)
                sc_i[...] = dl
                ndl = plsc.load_gather(sc_i, [jnp.minimum(iota + 1,
                                                          LANES - 1)])
                is_end = (dl != ndl) | (iota == LANES - 1)
                sc_f[...] = jnp.where(is_end, pcs, 0.0)
                zs = plsc.load_gather(sc_f, [jnp.maximum(iota - 1, 0)])
                zs = jnp.where(iota == 0, 0.0, zs)
                seg = pcs - plsc.cummax(zs)
                plsc.addupdate_scatter(den, [dl], seg, mask=is_end)
                return 0
            lax.fori_loop(0, nch, ch1, 0)
            return 0
        lax.fori_loop(0, (cend - cstart + SUP - 1) // SUP, sup1, 0)

        def gat_rows(o, buf, sem):
            pltpu.async_copy(
                h_hbm.at[st_src.at[pl.ds(o * LANES, LANES)]], buf, sem)

        def wait_rows(o, buf, sem):
            pltpu.make_async_copy(
                h_hbm.at[st_src.at[pl.ds(o * LANES, LANES)]], buf,
                sem).wait()

        def process(ch, o, buf, dlp, regs):
            _, valid, dlr, dl0, ex = edge_vecs(ch, o)
            dg = plsc.load_gather(den, [jnp.where(valid, dlr, npt)])
            alpha = ex / (dg + 1e-16)
            for k in range(LANES):
                a_k = alpha[k]
                d_k = dl0[k]
                diff = d_k != dlp

                @pl.when(diff)
                def _(regs=regs, dlp=dlp):
                    for c in range(dch):
                        plsc.addupdate(
                            acc.at[dlp, pl.ds(c * LANES, LANES)], regs[c])

                regs = tuple(
                    jnp.where(diff, 0.0, regs[c])
                    + a_k * buf[k, pl.ds(c * LANES, LANES)]
                    for c in range(dch))
                dlp = d_k
            return dlp, regs

        bufs = ((rows0, sem0), (rows1, sem1), (rows2, sem2), (rows3, sem3))

        def sup2(j, carry):
            bch = cstart + j * SUP
            nch = jnp.minimum(SUP, cend - bch)
            pltpu.sync_copy(
                srcs_hbm.at[pl.ds(bch * LANES, (SUP + 8) * LANES)], st_src)
            pltpu.sync_copy(
                dsts_hbm.at[pl.ds(bch * LANES, (SUP + 8) * LANES)], st_dst)

            nquad = (nch + 3) // 4
            nproc = 4 * nquad
            for i in range(3):
                gat_rows(i, bufs[i][0], bufs[i][1])

            def quad(q, carry):
                dlp, regs = carry[0], carry[1:]
                c0 = 4 * q
                for i in range(4):
                    ci = c0 + i
                    if i == 0:
                        gat_rows(c0 + 3, rows3, sem3)
                    else:
                        nxt = c0 + 3 + i

                        @pl.when(nxt < nproc)
                        def _(nxt=nxt, i=i):
                            gat_rows(nxt, bufs[(i - 1) % 4][0],
                                     bufs[(i - 1) % 4][1])
                    wait_rows(ci, bufs[i][0], bufs[i][1])
                    dlp, regs = process(bch + ci, ci, bufs[i][0], dlp,
                                        regs)
                return (dlp,) + regs
            return lax.fori_loop(0, nquad, quad, carry)

        zregs = tuple(jnp.zeros((LANES,), F32) for _ in range(dch))
        fin = lax.fori_loop(0, (cend - cstart + SUP - 1) // SUP, sup2,
                            (jnp.int32(npt),) + zregs)
        dlp_f, regs_f = fin[0], fin[1:]
        for c in range(dch):
            plsc.addupdate(acc.at[dlp_f, pl.ds(c * LANES, LANES)],
                           regs_f[c])

        pltpu.sync_copy(acc.at[pl.ds(0, npt)],
                        out_hbm.at[pl.ds(base_node, npt)])

    return sc_gat


def kernel(x, edge_index, params):
    n = x.shape[0]
    e = edge_index.shape[1]
    npt = ((n + NW - 1) // NW + 7) // 8 * 8
    np_ = npt * NW
    e1 = e + n
    ep = e1 + (SUP + 8) * LANES

    idx = edge_index.astype(I32)
    loop = jnp.arange(n, dtype=I32)
    src = jnp.concatenate([idx[0], loop])
    dst = jnp.concatenate([idx[1], loop])
    src_bits = max((n - 1).bit_length(), 1)
    key = (dst.astype(jnp.uint32) << src_bits) | src.astype(jnp.uint32)
    skey = jnp.sort(key)
    dsts = (skey >> src_bits).astype(I32)
    srcs = (skey & jnp.uint32((1 << src_bits) - 1)).astype(I32)
    srcs_p = jnp.concatenate([srcs, jnp.zeros((ep - e1,), I32)])
    dsts_p = jnp.concatenate([dsts, jnp.full((ep - e1,), np_ - 1, I32)])
    tgt = jnp.arange(NW + 1, dtype=I32) * npt
    epos = jnp.searchsorted(dsts, tgt).astype(I32)
    cstart = epos[:NW] // LANES
    cend = (epos[1:] + LANES - 1) // LANES
    bounds = jnp.concatenate([cstart, cend, jnp.zeros((LANES,), I32)])

    xp = jnp.pad(x.astype(F32), ((0, np_ - n), (0, 0)))

    sc_128 = _make_sc_gat(np_, 128, ep, npt, e1)
    sc_256 = _make_sc_gat(np_, 256, ep, npt, e1)

    def gat(y, p, bias_prev, relu_prev):
        w, a_s, a_d, b = p
        h, s, d = _proj(y, w, a_s, a_d, bias_prev, relu_prev)
        sc = sc_128 if w.shape[1] == 128 else sc_256
        msg = sc(h, s.reshape(np_), d.reshape(np_), srcs_p, dsts_p, bounds)
        return msg, b

    w1, b1 = params["lin1"]
    x1 = _lin1(xp, w1, b1)

    m, bg = gat(xp, params["gat1"][0], None, False)
    m, bg = gat(m, params["gat1"][1], bg, True)
    m, bg = gat(m, params["gat1"][2], bg, True)

    w2, b2 = params["lin2"]
    x2 = _resid(_mid_body, m, bg, w2, b2, x1)

    m, bg = gat(x2, params["gat2"][0], None, False)
    m, bg = gat(m, params["gat2"][1], bg, True)
    m, bg = gat(m, params["gat2"][2], bg, True)

    w3, b3 = params["lin3"]
    out = _resid(_fin_body, m, bg, w3, b3, x2)
    return out[:n]


# speedup vs baseline: 1.1302x; 1.1302x over previous
import functools

import jax
import jax.numpy as jnp
from jax import lax
from jax.experimental import pallas as pl
from jax.experimental.pallas import tpu as pltpu
from jax.experimental.pallas import tpu_sc as plsc

F32 = jnp.float32
I32 = jnp.int32

NW = 32
LANES = 16
SUP = 128
TC_BLK = 1280


def _proj_body(y_ref, w_ref, as_ref, ad_ref, b_ref, h_ref, s_ref, d_ref,
               *, kdim, relu, bias):
    y = y_ref[...]
    if bias:
        y = y + b_ref[...]
    if relu:
        y = jnp.maximum(y, 0.0)
    if kdim == 1:
        h = y * w_ref[...]
    else:
        h = jnp.dot(y, w_ref[...], preferred_element_type=F32)
    h_ref[...] = h
    s_ref[...] = jnp.sum(h * as_ref[...], axis=1, keepdims=True)
    d_ref[...] = jnp.sum(h * ad_ref[...], axis=1, keepdims=True)


def _proj(y, w, a_s, a_d, bias, relu):
    np_, k = y.shape
    m = w.shape[1]
    b2 = (bias if bias is not None else jnp.zeros((k,), F32)).reshape(1, k)
    body = functools.partial(_proj_body, kdim=k, relu=relu,
                             bias=bias is not None)
    return pl.pallas_call(
        body,
        grid=(np_ // TC_BLK,),
        in_specs=[
            pl.BlockSpec((TC_BLK, k), lambda i: (i, 0)),
            pl.BlockSpec((k, m), lambda i: (0, 0)),
            pl.BlockSpec((1, m), lambda i: (0, 0)),
            pl.BlockSpec((1, m), lambda i: (0, 0)),
            pl.BlockSpec((1, k), lambda i: (0, 0)),
        ],
        out_specs=[
            pl.BlockSpec((TC_BLK, m), lambda i: (i, 0)),
            pl.BlockSpec((TC_BLK, 1), lambda i: (i, 0)),
            pl.BlockSpec((TC_BLK, 1), lambda i: (i, 0)),
        ],
        out_shape=[
            jax.ShapeDtypeStruct((np_, m), F32),
            jax.ShapeDtypeStruct((np_, 1), F32),
            jax.ShapeDtypeStruct((np_, 1), F32),
        ],
    )(y, w, a_s.reshape(1, m), a_d.reshape(1, m), b2)


def _lin1_body(x_ref, w_ref, b_ref, o_ref):
    o_ref[...] = x_ref[...] * w_ref[...] + b_ref[...]


def _lin1(x, w, b):
    np_ = x.shape[0]
    m = w.shape[1]
    return pl.pallas_call(
        _lin1_body,
        grid=(np_ // TC_BLK,),
        in_specs=[
            pl.BlockSpec((TC_BLK, 1), lambda i: (i, 0)),
            pl.BlockSpec((1, m), lambda i: (0, 0)),
            pl.BlockSpec((1, m), lambda i: (0, 0)),
        ],
        out_specs=pl.BlockSpec((TC_BLK, m), lambda i: (i, 0)),
        out_shape=jax.ShapeDtypeStruct((np_, m), F32),
    )(x, w, b.reshape(1, m))


def _mid_body(m_ref, bg_ref, w_ref, b2_ref, x1_ref, o_ref):
    t = m_ref[...] + bg_ref[...]
    t = jnp.dot(t, w_ref[...], preferred_element_type=F32) + b2_ref[...]
    t = t + x1_ref[...]
    o_ref[...] = jnp.where(t >= 0, t, 0.01 * t)


def _fin_body(m_ref, bg_ref, w_ref, b3_ref, x2_ref, o_ref):
    t = m_ref[...] + bg_ref[...]
    t = jnp.dot(t, w_ref[...], preferred_element_type=F32) + b3_ref[...]
    o_ref[...] = jax.nn.sigmoid(x2_ref[...] + t)


def _resid(body, msg, bg, w, b, xr):
    np_, k = msg.shape
    m = w.shape[1]
    return pl.pallas_call(
        body,
        grid=(np_ // TC_BLK,),
        in_specs=[
            pl.BlockSpec((TC_BLK, k), lambda i: (i, 0)),
            pl.BlockSpec((1, k), lambda i: (0, 0)),
            pl.BlockSpec((k, m), lambda i: (0, 0)),
            pl.BlockSpec((1, m), lambda i: (0, 0)),
            pl.BlockSpec((TC_BLK, m), lambda i: (i, 0)),
        ],
        out_specs=pl.BlockSpec((TC_BLK, m), lambda i: (i, 0)),
        out_shape=jax.ShapeDtypeStruct((np_, m), F32),
    )(msg, bg.reshape(1, k), w, b.reshape(1, m), xr)


@functools.lru_cache(maxsize=None)
def _make_sc_gat(np_, d, ep, npt, e_real):
    mesh = plsc.VectorSubcoreMesh(core_axis_name="c", subcore_axis_name="s")
    dch = d // LANES

    @functools.partial(
        pl.kernel,
        mesh=mesh,
        out_type=jax.ShapeDtypeStruct((np_, d), F32),
        compiler_params=pltpu.CompilerParams(needs_layout_passes=False),
        scratch_types=[
            pltpu.VMEM((np_,), F32),
            pltpu.VMEM((npt,), F32),
            pltpu.VMEM((npt + LANES,), F32),
            pltpu.VMEM((npt + 1, d), F32),
            pltpu.VMEM(((SUP + 8) * LANES,), I32),
            pltpu.VMEM(((SUP + 8) * LANES,), I32),
            pltpu.VMEM((LANES, d), F32),
            pltpu.VMEM((LANES, d), F32),
            pltpu.VMEM((LANES,), I32),
            pltpu.VMEM((LANES,), F32),
            pltpu.VMEM((2 * NW + LANES,), I32),
            pltpu.SemaphoreType.DMA,
            pltpu.SemaphoreType.DMA,
        ],
    )
    def sc_gat(h_hbm, s_hbm, d_hbm, srcs_hbm, dsts_hbm, bounds_hbm,
               out_hbm, s_all, d_loc, den, acc, st_src, st_dst,
               rows0, rows1, sc_i, sc_f, bounds_v, sem0, sem1):
        w = lax.axis_index("s") * 2 + lax.axis_index("c")
        base_node = w * npt
        pltpu.sync_copy(bounds_hbm, bounds_v)
        cstart = bounds_v[pl.ds(w, LANES)][0]
        cend = bounds_v[pl.ds(NW + w, LANES)][0]
        pltpu.sync_copy(s_hbm, s_all)
        pltpu.sync_copy(d_hbm.at[pl.ds(base_node, npt)], d_loc)

        iota = lax.iota(I32, LANES)
        zero16 = jnp.zeros((LANES,), F32)

        def redm(i, m):
            return jnp.maximum(m, s_all[pl.ds(i * LANES, LANES)])
        mvec = lax.fori_loop(0, np_ // LANES, redm,
                             jnp.full((LANES,), -3.4e38, F32))
        smax = jnp.max(mvec)

        def zden(i, _):
            den[pl.ds(i * LANES, LANES)] = zero16
            return 0
        lax.fori_loop(0, (npt + LANES) // LANES, zden, 0)

        def zacc(r, _):
            for c in range(dch):
                acc[r, pl.ds(c * LANES, LANES)] = zero16
            return 0
        lax.fori_loop(0, npt + 1, zacc, 0)

        def edge_vecs(ch, o):
            sv = st_src[pl.ds(o * LANES, LANES)]
            dv = st_dst[pl.ds(o * LANES, LANES)]
            eid = ch * LANES + iota
            dlr = dv - base_node
            valid = (dlr >= 0) & (dlr < npt) & (eid < e_real)
            dl0 = jnp.where(valid, dlr, 0)
            ssrc = plsc.load_gather(s_all, [sv])
            dd = plsc.load_gather(d_loc, [dl0])
            mm = jnp.maximum(smax + dd, 0.0)
            t = ssrc + dd
            e = jnp.where(t >= 0, t, 0.2 * t)
            ex = jnp.where(valid, jnp.exp(e - mm), 0.0)
            return sv, valid, dlr, dl0, ex

        def sup1(j, _):
            bch = cstart + j * SUP
            nch = jnp.minimum(SUP, cend - bch)
            pltpu.sync_copy(
                srcs_hbm.at[pl.ds(bch * LANES, (SUP + 8) * LANES)], st_src)
            pltpu.sync_copy(
                dsts_hbm.at[pl.ds(bch * LANES, (SUP + 8) * LANES)], st_dst)

            def ch1(o, _):
                _, valid, dlr, _, ex = edge_vecs(bch + o, o)
                dl = jnp.where(valid, dlr, npt)
                pcs = plsc.cumsum(ex)
                sc_i[...] = dl
                ndl = plsc.load_gather(sc_i, [jnp.minimum(iota + 1,
                                                          LANES - 1)])
                is_end = (dl != ndl) | (iota == LANES - 1)
                sc_f[...] = jnp.where(is_end, pcs, 0.0)
                zs = plsc.load_gather(sc_f, [jnp.maximum(iota - 1, 0)])
                zs = jnp.where(iota == 0, 0.0, zs)
                seg = pcs - plsc.cummax(zs)
                plsc.addupdate_scatter(den, [dl], seg, mask=is_end)
                return 0
            lax.fori_loop(0, nch, ch1, 0)
            return 0
        lax.fori_loop(0, (cend - cstart + SUP - 1) // SUP, sup1, 0)

        def gat_rows(o, buf, sem):
            pltpu.async_copy(
                h_hbm.at[st_src.at[pl.ds(o * LANES, LANES)]], buf, sem)

        def wait_rows(o, buf, sem):
            pltpu.make_async_copy(
                h_hbm.at[st_src.at[pl.ds(o * LANES, LANES)]], buf,
                sem).wait()

        def process(ch, o, buf, dlp, regs):
            _, valid, dlr, dl0, ex = edge_vecs(ch, o)
            dg = plsc.load_gather(den, [jnp.where(valid, dlr, npt)])
            alpha = ex / (dg + 1e-16)
            for k in range(LANES):
                a_k = alpha[k]
                d_k = dl0[k]
                diff = d_k != dlp

                @pl.when(diff)
                def _(regs=regs, dlp=dlp):
                    for c in range(dch):
                        plsc.addupdate(
                            acc.at[dlp, pl.ds(c * LANES, LANES)], regs[c])

                regs = tuple(
                    jnp.where(diff, 0.0, regs[c])
                    + a_k * buf[k, pl.ds(c * LANES, LANES)]
                    for c in range(dch))
                dlp = d_k
            return dlp, regs

        def sup2(j, carry):
            bch = cstart + j * SUP
            nch = jnp.minimum(SUP, cend - bch)
            pltpu.sync_copy(
                srcs_hbm.at[pl.ds(bch * LANES, (SUP + 8) * LANES)], st_src)
            pltpu.sync_copy(
                dsts_hbm.at[pl.ds(bch * LANES, (SUP + 8) * LANES)], st_dst)

            npair = (nch + 1) // 2
            gat_rows(0, rows0, sem0)

            def pair(p, carry):
                dlp, regs = carry[0], carry[1:]
                c0 = 2 * p
                c1 = c0 + 1
                gat_rows(c1, rows1, sem1)
                wait_rows(c0, rows0, sem0)
                dlp, regs = process(bch + c0, c0, rows0, dlp, regs)

                @pl.when(p + 1 < npair)
                def _():
                    gat_rows(c0 + 2, rows0, sem0)

                wait_rows(c1, rows1, sem1)
                dlp, regs = process(bch + c1, c1, rows1, dlp, regs)
                return (dlp,) + regs
            return lax.fori_loop(0, npair, pair, carry)

        zregs = tuple(jnp.zeros((LANES,), F32) for _ in range(dch))
        fin = lax.fori_loop(0, (cend - cstart + SUP - 1) // SUP, sup2,
                            (jnp.int32(npt),) + zregs)
        dlp_f, regs_f = fin[0], fin[1:]
        for c in range(dch):
            plsc.addupdate(acc.at[dlp_f, pl.ds(c * LANES, LANES)],
                           regs_f[c])

        pltpu.sync_copy(acc.at[pl.ds(0, npt)],
                        out_hbm.at[pl.ds(base_node, npt)])

    return sc_gat


def kernel(x, edge_index, params):
    n = x.shape[0]
    e = edge_index.shape[1]
    npt = ((n + NW - 1) // NW + 7) // 8 * 8
    np_ = npt * NW
    e1 = e + n
    ep = e1 + (SUP + 8) * LANES

    idx = edge_index.astype(I32)
    loop = jnp.arange(n, dtype=I32)
    src = jnp.concatenate([idx[0], loop])
    dst = jnp.concatenate([idx[1], loop])
    src_bits = max((n - 1).bit_length(), 1)
    key = (dst.astype(jnp.uint32) << src_bits) | src.astype(jnp.uint32)
    skey = jnp.sort(key)
    dsts = (skey >> src_bits).astype(I32)
    srcs = (skey & jnp.uint32((1 << src_bits) - 1)).astype(I32)
    srcs_p = jnp.concatenate([srcs, jnp.zeros((ep - e1,), I32)])
    dsts_p = jnp.concatenate([dsts, jnp.full((ep - e1,), np_ - 1, I32)])
    tgt = jnp.arange(NW + 1, dtype=I32) * npt
    epos = jnp.searchsorted(dsts, tgt).astype(I32)
    cstart = epos[:NW] // LANES
    cend = (epos[1:] + LANES - 1) // LANES
    bounds = jnp.concatenate([cstart, cend, jnp.zeros((LANES,), I32)])

    xp = jnp.pad(x.astype(F32), ((0, np_ - n), (0, 0)))

    sc_128 = _make_sc_gat(np_, 128, ep, npt, e1)
    sc_256 = _make_sc_gat(np_, 256, ep, npt, e1)

    def gat(y, p, bias_prev, relu_prev):
        w, a_s, a_d, b = p
        h, s, d = _proj(y, w, a_s, a_d, bias_prev, relu_prev)
        sc = sc_128 if w.shape[1] == 128 else sc_256
        msg = sc(h, s.reshape(np_), d.reshape(np_), srcs_p, dsts_p, bounds)
        return msg, b

    w1, b1 = params["lin1"]
    x1 = _lin1(xp, w1, b1)

    m, bg = gat(xp, params["gat1"][0], None, False)
    m, bg = gat(m, params["gat1"][1], bg, True)
    m, bg = gat(m, params["gat1"][2], bg, True)

    w2, b2 = params["lin2"]
    x2 = _resid(_mid_body, m, bg, w2, b2, x1)

    m, bg = gat(x2, params["gat2"][0], None, False)
    m, bg = gat(m, params["gat2"][1], bg, True)
    m, bg = gat(m, params["gat2"][2], bg, True)

    w3, b3 = params["lin3"]
    out = _resid(_fin_body, m, bg, w3, b3, x2)
    return out[:n]


# speedup vs baseline: 1.2395x; 1.0967x over previous
import functools

import jax
import jax.numpy as jnp
from jax import lax
from jax.experimental import pallas as pl
from jax.experimental.pallas import tpu as pltpu
from jax.experimental.pallas import tpu_sc as plsc

F32 = jnp.float32
I32 = jnp.int32

NW = 32
LANES = 16
SUP = 128
TC_BLK = 1280


def _proj_body(y_ref, w_ref, as_ref, ad_ref, b_ref, h_ref, s_ref, d_ref,
               *, kdim, relu, bias):
    y = y_ref[...]
    if bias:
        y = y + b_ref[...]
    if relu:
        y = jnp.maximum(y, 0.0)
    if kdim == 1:
        h = y * w_ref[...]
    else:
        h = jnp.dot(y, w_ref[...], preferred_element_type=F32)
    h_ref[...] = h
    s_ref[...] = jnp.sum(h * as_ref[...], axis=1, keepdims=True)
    d_ref[...] = jnp.sum(h * ad_ref[...], axis=1, keepdims=True)


def _proj(y, w, a_s, a_d, bias, relu):
    np_, k = y.shape
    m = w.shape[1]
    b2 = (bias if bias is not None else jnp.zeros((k,), F32)).reshape(1, k)
    body = functools.partial(_proj_body, kdim=k, relu=relu,
                             bias=bias is not None)
    return pl.pallas_call(
        body,
        grid=(np_ // TC_BLK,),
        in_specs=[
            pl.BlockSpec((TC_BLK, k), lambda i: (i, 0)),
            pl.BlockSpec((k, m), lambda i: (0, 0)),
            pl.BlockSpec((1, m), lambda i: (0, 0)),
            pl.BlockSpec((1, m), lambda i: (0, 0)),
            pl.BlockSpec((1, k), lambda i: (0, 0)),
        ],
        out_specs=[
            pl.BlockSpec((TC_BLK, m), lambda i: (i, 0)),
            pl.BlockSpec((TC_BLK, 1), lambda i: (i, 0)),
            pl.BlockSpec((TC_BLK, 1), lambda i: (i, 0)),
        ],
        out_shape=[
            jax.ShapeDtypeStruct((np_, m), F32),
            jax.ShapeDtypeStruct((np_, 1), F32),
            jax.ShapeDtypeStruct((np_, 1), F32),
        ],
    )(y, w, a_s.reshape(1, m), a_d.reshape(1, m), b2)


def _lin1_body(x_ref, w_ref, b_ref, o_ref):
    o_ref[...] = x_ref[...] * w_ref[...] + b_ref[...]


def _lin1(x, w, b):
    np_ = x.shape[0]
    m = w.shape[1]
    return pl.pallas_call(
        _lin1_body,
        grid=(np_ // TC_BLK,),
        in_specs=[
            pl.BlockSpec((TC_BLK, 1), lambda i: (i, 0)),
            pl.BlockSpec((1, m), lambda i: (0, 0)),
            pl.BlockSpec((1, m), lambda i: (0, 0)),
        ],
        out_specs=pl.BlockSpec((TC_BLK, m), lambda i: (i, 0)),
        out_shape=jax.ShapeDtypeStruct((np_, m), F32),
    )(x, w, b.reshape(1, m))


def _mid_body(m_ref, bg_ref, w_ref, b2_ref, x1_ref, o_ref):
    t = m_ref[...] + bg_ref[...]
    t = jnp.dot(t, w_ref[...], preferred_element_type=F32) + b2_ref[...]
    t = t + x1_ref[...]
    o_ref[...] = jnp.where(t >= 0, t, 0.01 * t)


def _fin_body(m_ref, bg_ref, w_ref, b3_ref, x2_ref, o_ref):
    t = m_ref[...] + bg_ref[...]
    t = jnp.dot(t, w_ref[...], preferred_element_type=F32) + b3_ref[...]
    o_ref[...] = jax.nn.sigmoid(x2_ref[...] + t)


def _resid(body, msg, bg, w, b, xr):
    np_, k = msg.shape
    m = w.shape[1]
    return pl.pallas_call(
        body,
        grid=(np_ // TC_BLK,),
        in_specs=[
            pl.BlockSpec((TC_BLK, k), lambda i: (i, 0)),
            pl.BlockSpec((1, k), lambda i: (0, 0)),
            pl.BlockSpec((k, m), lambda i: (0, 0)),
            pl.BlockSpec((1, m), lambda i: (0, 0)),
            pl.BlockSpec((TC_BLK, m), lambda i: (i, 0)),
        ],
        out_specs=pl.BlockSpec((TC_BLK, m), lambda i: (i, 0)),
        out_shape=jax.ShapeDtypeStruct((np_, m), F32),
    )(msg, bg.reshape(1, k), w, b.reshape(1, m), xr)


@functools.lru_cache(maxsize=None)
def _make_sc_gat(np_, d, ep, npt, e_real):
    mesh = plsc.VectorSubcoreMesh(core_axis_name="c", subcore_axis_name="s")
    dch = d // LANES

    @functools.partial(
        pl.kernel,
        mesh=mesh,
        out_type=jax.ShapeDtypeStruct((np_, d), F32),
        compiler_params=pltpu.CompilerParams(needs_layout_passes=False),
        scratch_types=[
            pltpu.VMEM((np_,), F32),
            pltpu.VMEM((npt,), F32),
            pltpu.VMEM((npt + LANES,), F32),
            pltpu.VMEM((npt + 1, d), F32),
            pltpu.VMEM(((SUP + 8) * LANES,), I32),
            pltpu.VMEM(((SUP + 8) * LANES,), I32),
            pltpu.VMEM((LANES, d), F32),
            pltpu.VMEM((LANES, d), F32),
            pltpu.VMEM((LANES,), I32),
            pltpu.VMEM((LANES,), F32),
            pltpu.VMEM((2 * NW + LANES,), I32),
            pltpu.SemaphoreType.DMA,
            pltpu.SemaphoreType.DMA,
        ],
    )
    def sc_gat(h_hbm, s_hbm, d_hbm, srcs_hbm, dsts_hbm, bounds_hbm,
               out_hbm, s_all, d_loc, den, acc, st_src, st_dst,
               rows0, rows1, sc_i, sc_f, bounds_v, sem0, sem1):
        w = lax.axis_index("s") * 2 + lax.axis_index("c")
        base_node = w * npt
        pltpu.sync_copy(bounds_hbm, bounds_v)
        cstart = bounds_v[pl.ds(w, LANES)][0]
        cend = bounds_v[pl.ds(NW + w, LANES)][0]
        pltpu.sync_copy(s_hbm, s_all)
        pltpu.sync_copy(d_hbm.at[pl.ds(base_node, npt)], d_loc)

        iota = lax.iota(I32, LANES)
        zero16 = jnp.zeros((LANES,), F32)

        def redm(i, m):
            return jnp.maximum(m, s_all[pl.ds(i * LANES, LANES)])
        mvec = lax.fori_loop(0, np_ // LANES, redm,
                             jnp.full((LANES,), -3.4e38, F32))
        smax = jnp.max(mvec)

        def zden(i, _):
            den[pl.ds(i * LANES, LANES)] = zero16
            return 0
        lax.fori_loop(0, (npt + LANES) // LANES, zden, 0)

        def zacc(r, _):
            for c in range(dch):
                acc[r, pl.ds(c * LANES, LANES)] = zero16
            return 0
        lax.fori_loop(0, npt + 1, zacc, 0)

        def edge_vecs(ch, o):
            sv = st_src[pl.ds(o * LANES, LANES)]
            dv = st_dst[pl.ds(o * LANES, LANES)]
            eid = ch * LANES + iota
            dlr = dv - base_node
            valid = (dlr >= 0) & (dlr < npt) & (eid < e_real)
            dl0 = jnp.where(valid, dlr, 0)
            ssrc = plsc.load_gather(s_all, [sv])
            dd = plsc.load_gather(d_loc, [dl0])
            mm = jnp.maximum(smax + dd, 0.0)
            t = ssrc + dd
            e = jnp.where(t >= 0, t, 0.2 * t)
            ex = jnp.where(valid, jnp.exp(e - mm), 0.0)
            return sv, valid, dlr, dl0, ex

        def sup1(j, _):
            bch = cstart + j * SUP
            nch = jnp.minimum(SUP, cend - bch)
            pltpu.sync_copy(
                srcs_hbm.at[pl.ds(bch * LANES, (SUP + 8) * LANES)], st_src)
            pltpu.sync_copy(
                dsts_hbm.at[pl.ds(bch * LANES, (SUP + 8) * LANES)], st_dst)

            def ch1(o, _):
                _, valid, dlr, _, ex = edge_vecs(bch + o, o)
                dl = jnp.where(valid, dlr, npt)
                plsc.addupdate_scatter(den, [dl], ex)
                return 0
            lax.fori_loop(0, nch, ch1, 0)
            return 0
        lax.fori_loop(0, (cend - cstart + SUP - 1) // SUP, sup1, 0)

        def gat_rows(o, buf, sem):
            pltpu.async_copy(
                h_hbm.at[st_src.at[pl.ds(o * LANES, LANES)]], buf, sem)

        def wait_rows(o, buf, sem):
            pltpu.make_async_copy(
                h_hbm.at[st_src.at[pl.ds(o * LANES, LANES)]], buf,
                sem).wait()

        def process(ch, o, buf, dlp, regs):
            _, valid, dlr, dl0, ex = edge_vecs(ch, o)
            dg = plsc.load_gather(den, [jnp.where(valid, dlr, npt)])
            alpha = ex / (dg + 1e-16)
            for k in range(LANES):
                a_k = alpha[k]
                d_k = dl0[k]
                diff = d_k != dlp

                @pl.when(diff)
                def _(regs=regs, dlp=dlp):
                    for c in range(dch):
                        plsc.addupdate(
                            acc.at[dlp, pl.ds(c * LANES, LANES)], regs[c])

                regs = tuple(
                    jnp.where(diff, 0.0, regs[c])
                    + a_k * buf[k, pl.ds(c * LANES, LANES)]
                    for c in range(dch))
                dlp = d_k
            return dlp, regs

        def sup2(j, carry):
            bch = cstart + j * SUP
            nch = jnp.minimum(SUP, cend - bch)
            pltpu.sync_copy(
                srcs_hbm.at[pl.ds(bch * LANES, (SUP + 8) * LANES)], st_src)
            pltpu.sync_copy(
                dsts_hbm.at[pl.ds(bch * LANES, (SUP + 8) * LANES)], st_dst)

            npair = (nch + 1) // 2
            gat_rows(0, rows0, sem0)

            def pair(p, carry):
                dlp, regs = carry[0], carry[1:]
                c0 = 2 * p
                c1 = c0 + 1
                gat_rows(c1, rows1, sem1)
                wait_rows(c0, rows0, sem0)
                dlp, regs = process(bch + c0, c0, rows0, dlp, regs)

                @pl.when(p + 1 < npair)
                def _():
                    gat_rows(c0 + 2, rows0, sem0)

                wait_rows(c1, rows1, sem1)
                dlp, regs = process(bch + c1, c1, rows1, dlp, regs)
                return (dlp,) + regs
            return lax.fori_loop(0, npair, pair, carry)

        zregs = tuple(jnp.zeros((LANES,), F32) for _ in range(dch))
        fin = lax.fori_loop(0, (cend - cstart + SUP - 1) // SUP, sup2,
                            (jnp.int32(npt),) + zregs)
        dlp_f, regs_f = fin[0], fin[1:]
        for c in range(dch):
            plsc.addupdate(acc.at[dlp_f, pl.ds(c * LANES, LANES)],
                           regs_f[c])

        pltpu.sync_copy(acc.at[pl.ds(0, npt)],
                        out_hbm.at[pl.ds(base_node, npt)])

    return sc_gat


def kernel(x, edge_index, params):
    n = x.shape[0]
    e = edge_index.shape[1]
    npt = ((n + NW - 1) // NW + 7) // 8 * 8
    np_ = npt * NW
    e1 = e + n
    ep = e1 + (SUP + 8) * LANES

    idx = edge_index.astype(I32)
    loop = jnp.arange(n, dtype=I32)
    src = jnp.concatenate([idx[0], loop])
    dst = jnp.concatenate([idx[1], loop])
    src_bits = max((n - 1).bit_length(), 1)
    key = (dst.astype(jnp.uint32) << src_bits) | src.astype(jnp.uint32)
    skey = lax.sort([key], is_stable=False)[0]
    dsts = (skey >> src_bits).astype(I32)
    srcs = (skey & jnp.uint32((1 << src_bits) - 1)).astype(I32)
    srcs_p = jnp.concatenate([srcs, jnp.zeros((ep - e1,), I32)])
    dsts_p = jnp.concatenate([dsts, jnp.full((ep - e1,), np_ - 1, I32)])
    tgt = jnp.arange(NW + 1, dtype=I32) * npt
    epos = jnp.searchsorted(dsts, tgt).astype(I32)
    cstart = epos[:NW] // LANES
    cend = (epos[1:] + LANES - 1) // LANES
    bounds = jnp.concatenate([cstart, cend, jnp.zeros((LANES,), I32)])

    xp = jnp.pad(x.astype(F32), ((0, np_ - n), (0, 0)))

    sc_128 = _make_sc_gat(np_, 128, ep, npt, e1)
    sc_256 = _make_sc_gat(np_, 256, ep, npt, e1)

    def gat(y, p, bias_prev, relu_prev):
        w, a_s, a_d, b = p
        h, s, d = _proj(y, w, a_s, a_d, bias_prev, relu_prev)
        sc = sc_128 if w.shape[1] == 128 else sc_256
        msg = sc(h, s.reshape(np_), d.reshape(np_), srcs_p, dsts_p, bounds)
        return msg, b

    w1, b1 = params["lin1"]
    x1 = _lin1(xp, w1, b1)

    m, bg = gat(xp, params["gat1"][0], None, False)
    m, bg = gat(m, params["gat1"][1], bg, True)
    m, bg = gat(m, params["gat1"][2], bg, True)

    w2, b2 = params["lin2"]
    x2 = _resid(_mid_body, m, bg, w2, b2, x1)

    m, bg = gat(x2, params["gat2"][0], None, False)
    m, bg = gat(m, params["gat2"][1], bg, True)
    m, bg = gat(m, params["gat2"][2], bg, True)

    w3, b3 = params["lin3"]
    out = _resid(_fin_body, m, bg, w3, b3, x2)
    return out[:n]
